# Initial kernel scaffold; baseline (speedup 1.0000x reference)
import jax
import jax.numpy as jnp
from jax.experimental import pallas as pl


def kernel(x, y, pos_id):
    raise NotImplementedError("write your pallas kernel here")


# speedup vs baseline: 1.9428x; 1.9428x over previous
import functools

import jax
import jax.numpy as jnp
from jax import lax
from jax.experimental import pallas as pl
from jax.experimental.pallas import tpu as pltpu
from jax.experimental.pallas import tpu_sc as plsc


def _sc_gather(table, idx):
    V, D = table.shape
    B = idx.shape[0]
    info = plsc.get_sparse_core_info()
    nw = info.num_cores * info.num_subcores
    b_per_w = B // nw
    mesh = plsc.VectorSubcoreMesh(core_axis_name="c", subcore_axis_name="s")

    @functools.partial(
        pl.kernel,
        mesh=mesh,
        out_type=jax.ShapeDtypeStruct((B, D), jnp.float32),
        scratch_types=[
            pltpu.VMEM((b_per_w,), jnp.int32),
            pltpu.VMEM((b_per_w, D), jnp.float32),
            pltpu.SemaphoreType.DMA,
        ],
    )
    def k(table_hbm, idx_hbm, out_hbm, idx_v, rows_v, sem):
        wid = lax.axis_index("s") * info.num_cores + lax.axis_index("c")
        base = wid * b_per_w
        pltpu.sync_copy(idx_hbm.at[pl.ds(base, b_per_w)], idx_v)
        pltpu.async_copy(table_hbm.at[idx_v], rows_v, sem).wait()
        pltpu.sync_copy(rows_v, out_hbm.at[pl.ds(base, b_per_w)])

    return k(table, idx)


def _mul_body(g_ref, y_ref, o_ref):
    o_ref[...] = g_ref[...][:, None, :] * y_ref[...][None, :, :]


def _tc_mul(g, ys):
    B, D = g.shape
    J = ys.shape[0]
    bg = 256
    return pl.pallas_call(
        _mul_body,
        grid=(B // bg,),
        in_specs=[
            pl.BlockSpec((bg, D), lambda i: (i, 0)),
            pl.BlockSpec((J, D), lambda i: (0, 0)),
        ],
        out_specs=pl.BlockSpec((bg, J, D), lambda i: (i, 0, 0)),
        out_shape=jax.ShapeDtypeStruct((B, J, D), jnp.float32),
    )(g, ys)


def kernel(x, y, pos_id):
    xs = x.reshape(x.shape[2], x.shape[3])
    ys = y.reshape(y.shape[1], y.shape[3])
    idx = pos_id.reshape(-1)
    g = _sc_gather(xs, idx)
    out = _tc_mul(g, ys)
    return out.reshape(out.shape[0], ys.shape[0], 1, xs.shape[1])


# speedup vs baseline: 1.9840x; 1.0212x over previous
import functools

import jax
import jax.numpy as jnp
from jax import lax
from jax.experimental import pallas as pl
from jax.experimental.pallas import tpu as pltpu
from jax.experimental.pallas import tpu_sc as plsc


def _sc_gather(table, idx):
    V, D = table.shape
    B = idx.shape[0]
    info = plsc.get_sparse_core_info()
    nw = info.num_cores * info.num_subcores
    b_per_w = B // nw
    mesh = plsc.VectorSubcoreMesh(core_axis_name="c", subcore_axis_name="s")

    @functools.partial(
        pl.kernel,
        mesh=mesh,
        out_type=jax.ShapeDtypeStruct((B, D), jnp.float32),
        scratch_types=[
            pltpu.VMEM((b_per_w,), jnp.int32),
            pltpu.VMEM((b_per_w, D), jnp.float32),
            pltpu.SemaphoreType.DMA,
        ],
    )
    def k(table_hbm, idx_hbm, out_hbm, idx_v, rows_v, sem):
        wid = lax.axis_index("s") * info.num_cores + lax.axis_index("c")
        base = wid * b_per_w
        pltpu.sync_copy(idx_hbm.at[pl.ds(base, b_per_w)], idx_v)
        pltpu.async_copy(table_hbm.at[idx_v], rows_v, sem).wait()
        pltpu.sync_copy(rows_v, out_hbm.at[pl.ds(base, b_per_w)])

    return k(table, idx)


def _mul_body(g_ref, y_ref, o_ref):
    o_ref[...] = g_ref[...][:, None, :] * y_ref[...][None, :, :]


def _tc_mul(g, ys):
    B, D = g.shape
    J = ys.shape[0]
    bg = 512
    return pl.pallas_call(
        _mul_body,
        grid=(B // bg,),
        in_specs=[
            pl.BlockSpec((bg, D), lambda i: (i, 0)),
            pl.BlockSpec((J, D), lambda i: (0, 0)),
        ],
        out_specs=pl.BlockSpec((bg, J, D), lambda i: (i, 0, 0)),
        out_shape=jax.ShapeDtypeStruct((B, J, D), jnp.float32),
    )(g, ys)


def kernel(x, y, pos_id):
    xs = x.reshape(x.shape[2], x.shape[3])
    ys = y.reshape(y.shape[1], y.shape[3])
    idx = pos_id.reshape(-1)
    g = _sc_gather(xs, idx)
    out = _tc_mul(g, ys)
    return out.reshape(out.shape[0], ys.shape[0], 1, xs.shape[1])
